# Initial kernel scaffold; baseline (speedup 1.0000x reference)
import jax
import jax.numpy as jnp
from jax.experimental import pallas as pl


def kernel(x, edge_index, pos, params):
    raise NotImplementedError("write your pallas kernel here")


# speedup vs baseline: 4.9331x; 4.9331x over previous
import functools

import jax
import jax.numpy as jnp
from jax import lax
from jax.experimental import pallas as pl
from jax.experimental.pallas import tpu as pltpu
from jax.experimental.pallas import tpu_sc as plsc

N = 10000
E = 320000
H = 128
NC = 2
NS = 16
NW = NC * NS
CH = 128
NCHUNK = E // CH
CHUNKS_PER_W = -(-NCHUNK // NW)
RPT = 624
RPT_LAST = N - 15 * RPT
ZR = 128

_K1 = -2.0 * 0.7978845608028654
_K2 = _K1 * 0.044715

_mesh = plsc.VectorSubcoreMesh(
    core_axis_name="c", subcore_axis_name="s", num_cores=NC, num_subcores=NS)


def _gelu16(t):
  d = jnp.exp(_K1 * t + _K2 * (t * t * t))
  return t / (1.0 + d)


@functools.partial(
    pl.kernel,
    out_type=jax.ShapeDtypeStruct((NC, N, H), jnp.float32),
    mesh=_mesh,
    scratch_types=[
        pltpu.VMEM((CH,), jnp.int32),
        pltpu.VMEM((CH, H), jnp.float32),
        pltpu.VMEM((ZR, H), jnp.float32),
        pltpu.VMEM_SHARED((N, H), jnp.float32),
    ],
)
def _sc_degree(dst2d, cnt_out, idx_v, ones_v, z16, cnt_sh):
  c = lax.axis_index("c")
  s = lax.axis_index("s")
  wid = s * NC + c

  @pl.loop(0, CH)
  def _(i):
    for k in range(H // 16):
      ones_v[i, pl.ds(k * 16, 16)] = jnp.ones((16,), jnp.float32)

  @pl.loop(0, ZR)
  def _(i):
    for k in range(H // 16):
      z16[i, pl.ds(k * 16, 16)] = jnp.zeros((16,), jnp.float32)

  for r5 in range(5):
    pltpu.sync_copy(z16, cnt_sh.at[pl.ds(s * RPT + r5 * ZR, ZR)])
  plsc.subcore_barrier()

  @pl.loop(0, CHUNKS_PER_W)
  def _(i):
    cidx = wid + i * NW

    @pl.when(cidx < NCHUNK)
    def _():
      pltpu.sync_copy(dst2d.at[cidx], idx_v)
      pltpu.sync_copy(ones_v, cnt_sh.at[idx_v], add=True)

  plsc.subcore_barrier()

  @pl.when(s < NS - 1)
  def _():
    pltpu.sync_copy(cnt_sh.at[pl.ds(s * RPT, RPT)],
                    cnt_out.at[c, pl.ds(s * RPT, RPT)])

  @pl.when(s == NS - 1)
  def _():
    pltpu.sync_copy(cnt_sh.at[pl.ds(15 * RPT, RPT_LAST)],
                    cnt_out.at[c, pl.ds(15 * RPT, RPT_LAST)])


@functools.partial(
    pl.kernel,
    out_type=jax.ShapeDtypeStruct((NC, N, H), jnp.float32),
    mesh=_mesh,
    scratch_types=[
        pltpu.VMEM((CH,), jnp.int32),
        pltpu.VMEM((CH,), jnp.int32),
        pltpu.VMEM((CH, H), jnp.float32),
        pltpu.VMEM((CH, H), jnp.float32),
        pltpu.VMEM((ZR, H), jnp.float32),
        pltpu.VMEM_SHARED((N, H), jnp.float32),
        pltpu.SemaphoreType.DMA,
        pltpu.SemaphoreType.DMA,
    ],
)
def _sc_messages(u_hbm, v_hbm, dst2d, src2d, g_out,
                 dst_v, src_v, u_buf, v_buf, zbuf, g_sh, sem_u, sem_v):
  c = lax.axis_index("c")
  s = lax.axis_index("s")
  wid = s * NC + c

  @pl.loop(0, ZR)
  def _(r):
    for k in range(H // 16):
      zbuf[r, pl.ds(k * 16, 16)] = jnp.zeros((16,), jnp.float32)

  for r5 in range(5):
    pltpu.sync_copy(zbuf, g_sh.at[pl.ds(s * RPT + r5 * ZR, ZR)])
  plsc.subcore_barrier()

  @pl.loop(0, CHUNKS_PER_W)
  def _(i):
    cidx = wid + i * NW

    @pl.when(cidx < NCHUNK)
    def _():
      pltpu.sync_copy(dst2d.at[cidx], dst_v)
      pltpu.sync_copy(src2d.at[cidx], src_v)
      cp_u = pltpu.async_copy(u_hbm.at[dst_v], u_buf, sem_u)
      cp_v = pltpu.async_copy(v_hbm.at[src_v], v_buf, sem_v)
      cp_u.wait()
      cp_v.wait()

      @pl.loop(0, CH)
      def _(r):
        for k in range(H // 16):
          sl = pl.ds(k * 16, 16)
          u_buf[r, sl] = _gelu16(u_buf[r, sl] + v_buf[r, sl])

      pltpu.sync_copy(u_buf, g_sh.at[dst_v], add=True)

  plsc.subcore_barrier()

  @pl.when(s < NS - 1)
  def _():
    pltpu.sync_copy(g_sh.at[pl.ds(s * RPT, RPT)],
                    g_out.at[c, pl.ds(s * RPT, RPT)])

  @pl.when(s == NS - 1)
  def _():
    pltpu.sync_copy(g_sh.at[pl.ds(15 * RPT, RPT_LAST)],
                    g_out.at[c, pl.ds(15 * RPT, RPT_LAST)])


R = 2000
GRID = N // R

def _full(shape):
  return pl.BlockSpec(shape, lambda i: tuple(0 for _ in shape))


def _tc_pre_body(x_r, pos_r, wi1, bi1, wi2, bi2, w1a, w1b, w1c, b1e,
                 h_r, u_r, v_r):
  f32 = jnp.float32
  h = jnp.dot(x_r[...], wi1[...], preferred_element_type=f32) + bi1[...]
  h = jnp.dot(jax.nn.gelu(h), wi2[...], preferred_element_type=f32) + bi2[...]
  p = jnp.dot(pos_r[...], w1c[...], preferred_element_type=f32)
  h_r[...] = h
  u_r[...] = jnp.dot(h, w1a[...], preferred_element_type=f32) - p + b1e[...]
  v_r[...] = jnp.dot(h, w1b[...], preferred_element_type=f32) + p


_tc_pre = pl.pallas_call(
    _tc_pre_body,
    grid=(GRID,),
    in_specs=[
        pl.BlockSpec((R, H), lambda i: (i, 0)),
        pl.BlockSpec((R, 8), lambda i: (i, 0)),
        _full((H, H)), _full((1, H)), _full((H, H)), _full((1, H)),
        _full((H, H)), _full((H, H)), _full((8, H)), _full((1, H)),
    ],
    out_specs=[
        pl.BlockSpec((R, H), lambda i: (i, 0)),
        pl.BlockSpec((R, H), lambda i: (i, 0)),
        pl.BlockSpec((R, H), lambda i: (i, 0)),
    ],
    out_shape=[jax.ShapeDtypeStruct((N, H), jnp.float32)] * 3,
)


def _node_update(h_r, g2_r, cnt2_r, w2e, b2e, wn1a, wn1b, b1n, wn2, b2n,
                 lng, lnb):
  f32 = jnp.float32
  g = g2_r[0] + g2_r[1]
  cnt = cnt2_r[0, :, 0:1] + cnt2_r[1, :, 0:1]
  aggs = jnp.dot(g, w2e[...], preferred_element_type=f32) + cnt * b2e[...]
  agg = aggs / jnp.maximum(cnt, 1.0)
  hh = h_r[...]
  o = (jnp.dot(hh, wn1a[...], preferred_element_type=f32)
       + jnp.dot(agg, wn1b[...], preferred_element_type=f32) + b1n[...])
  o = jnp.dot(jax.nn.gelu(o), wn2[...], preferred_element_type=f32) + b2n[...]
  hr = hh + o
  mu = jnp.mean(hr, axis=-1, keepdims=True)
  dif = hr - mu
  var = jnp.mean(dif * dif, axis=-1, keepdims=True)
  return dif * jax.lax.rsqrt(var + 1e-5) * lng[...] + lnb[...]


def _tc_mid_body(h_r, g2_r, cnt2_r, pos_r, w2e, b2e, wn1a, wn1b, b1n, wn2,
                 b2n, lng, lnb, w1a, w1b, w1c, b1e, h_out, u_out, v_out):
  f32 = jnp.float32
  hn = _node_update(h_r, g2_r, cnt2_r, w2e, b2e, wn1a, wn1b, b1n, wn2, b2n,
                    lng, lnb)
  p = jnp.dot(pos_r[...], w1c[...], preferred_element_type=f32)
  h_out[...] = hn
  u_out[...] = jnp.dot(hn, w1a[...], preferred_element_type=f32) - p + b1e[...]
  v_out[...] = jnp.dot(hn, w1b[...], preferred_element_type=f32) + p


_tc_mid = pl.pallas_call(
    _tc_mid_body,
    grid=(GRID,),
    in_specs=[
        pl.BlockSpec((R, H), lambda i: (i, 0)),
        pl.BlockSpec((NC, R, H), lambda i: (0, i, 0)),
        pl.BlockSpec((NC, R, 8), lambda i: (0, i, 0)),
        pl.BlockSpec((R, 8), lambda i: (i, 0)),
        _full((H, H)), _full((1, H)), _full((H, H)), _full((H, H)),
        _full((1, H)), _full((H, H)), _full((1, H)), _full((1, H)),
        _full((1, H)),
        _full((H, H)), _full((H, H)), _full((8, H)), _full((1, H)),
    ],
    out_specs=[
        pl.BlockSpec((R, H), lambda i: (i, 0)),
        pl.BlockSpec((R, H), lambda i: (i, 0)),
        pl.BlockSpec((R, H), lambda i: (i, 0)),
    ],
    out_shape=[jax.ShapeDtypeStruct((N, H), jnp.float32)] * 3,
)


def _tc_final_body(h_r, g2_r, cnt2_r, w2e, b2e, wn1a, wn1b, b1n, wn2, b2n,
                   lng, lnb, wh1, bh1, wh2, bh2, pred_out):
  f32 = jnp.float32
  hn = _node_update(h_r, g2_r, cnt2_r, w2e, b2e, wn1a, wn1b, b1n, wn2, b2n,
                    lng, lnb)
  q = jnp.dot(hn, wh1[...], preferred_element_type=f32) + bh1[...]
  q = jax.nn.gelu(q)
  pred_out[...] = jnp.dot(q, wh2[...], preferred_element_type=f32) + bh2[...]


_tc_final = pl.pallas_call(
    _tc_final_body,
    grid=(GRID,),
    in_specs=[
        pl.BlockSpec((R, H), lambda i: (i, 0)),
        pl.BlockSpec((NC, R, H), lambda i: (0, i, 0)),
        pl.BlockSpec((NC, R, 8), lambda i: (0, i, 0)),
        _full((H, H)), _full((1, H)), _full((H, H)), _full((H, H)),
        _full((1, H)), _full((H, H)), _full((1, H)), _full((1, H)),
        _full((1, H)),
        _full((H, H)), _full((1, H)), _full((H, 1)), _full((1, 1)),
    ],
    out_specs=[pl.BlockSpec((R, 1), lambda i: (i, 0))],
    out_shape=[jax.ShapeDtypeStruct((N, 1), jnp.float32)],
)


def _row(v):
  return v.reshape(1, -1)


def kernel(x, edge_index, pos, params):
  dst2d = edge_index[1].reshape(NCHUNK, CH)
  src2d = edge_index[0].reshape(NCHUNK, CH)
  pos8 = jnp.pad(pos, ((0, 0), (0, 6)))

  cnt2 = _sc_degree(dst2d)[:, :, :8]

  blocks = params["blocks"]

  def edge_w(blk):
    w1 = blk["edge"]["W1"]
    w1c8 = jnp.pad(w1[2 * H:], ((0, 6), (0, 0)))
    return w1[:H], w1[H:2 * H], w1c8, _row(blk["edge"]["b1"])

  ip = params["input_proj"]
  w1a, w1b, w1c8, b1e = edge_w(blocks[0])
  h, u, v = _tc_pre(x, pos8, ip["W1"], _row(ip["b1"]), ip["W2"],
                    _row(ip["b2"]), w1a, w1b, w1c8, b1e)

  for s in range(len(blocks)):
    blk = blocks[s]
    g2 = _sc_messages(u, v, dst2d, src2d)
    nw = blk["node"]
    step_w = (blk["edge"]["W2"], _row(blk["edge"]["b2"]),
              nw["W1"][:H], nw["W1"][H:], _row(nw["b1"]), nw["W2"],
              _row(nw["b2"]), _row(blk["ln_g"]), _row(blk["ln_b"]))
    if s + 1 < len(blocks):
      w1a, w1b, w1c8, b1e = edge_w(blocks[s + 1])
      h, u, v = _tc_mid(h, g2, cnt2, pos8, *step_w, w1a, w1b, w1c8, b1e)
    else:
      hd = params["head"]
      (pred,) = _tc_final(h, g2, cnt2, *step_w, hd["W1"], _row(hd["b1"]),
                          hd["W2"], _row(hd["b2"]))
  return pred
